# baseline (device time: 2190755 ns/iter reference)
import jax
import jax.numpy as jnp
from jax import lax
from jax.experimental import pallas as pl
from jax.experimental.pallas import tpu as pltpu

K = 16


def kernel(x):
    x = x.astype(jnp.bfloat16)
    m, n = x.shape
    h = m // 2
    r = h // K

    def body(x_ref, out_ref, local_sem, x_send_sems, x_recv_sems,
             y_send_sems, y_recv_sems):
        my_x = lax.axis_index("x")
        my_y = lax.axis_index("y")
        other_x = 1 - my_x
        other_y = 1 - my_y
        base_mine = my_x * m
        base_rem = other_x * m

        barrier_sem = pltpu.get_barrier_semaphore()
        for nbr in [(other_x, my_y), (my_x, other_y)]:
            pl.semaphore_signal(
                barrier_sem, inc=1,
                device_id=nbr, device_id_type=pl.DeviceIdType.MESH,
            )
        pl.semaphore_wait(barrier_sem, 2)

        local_copy = pltpu.make_async_copy(
            x_ref, out_ref.at[pl.ds(base_mine, m)], local_sem
        )
        local_copy.start()

        x_sends = []
        for c in range(K):
            off = my_y * h + c * r
            s = pltpu.make_async_remote_copy(
                src_ref=x_ref.at[pl.ds(off, r)],
                dst_ref=out_ref.at[pl.ds(base_mine + my_y * h + c * r, r)],
                send_sem=x_send_sems.at[c],
                recv_sem=x_recv_sems.at[c],
                device_id=(other_x, my_y),
                device_id_type=pl.DeviceIdType.MESH,
            )
            s.start()
            x_sends.append(s)

        y_sends = []
        for c in range(K):
            rows = pl.ds(base_rem + my_y * h + c * r, r)
            recv = pltpu.make_async_remote_copy(
                src_ref=x_ref.at[pl.ds(c * r, r)],
                dst_ref=out_ref.at[rows],
                send_sem=x_send_sems.at[c],
                recv_sem=x_recv_sems.at[c],
                device_id=(other_x, my_y),
                device_id_type=pl.DeviceIdType.MESH,
            )
            recv.wait_recv()
            f = pltpu.make_async_remote_copy(
                src_ref=out_ref.at[rows],
                dst_ref=out_ref.at[rows],
                send_sem=y_send_sems.at[c],
                recv_sem=y_recv_sems.at[c],
                device_id=(my_x, other_y),
                device_id_type=pl.DeviceIdType.MESH,
            )
            f.start()
            y_sends.append(f)

        for c in range(K):
            recv = pltpu.make_async_remote_copy(
                src_ref=x_ref.at[pl.ds(c * r, r)],
                dst_ref=out_ref.at[pl.ds(base_rem + other_y * h + c * r, r)],
                send_sem=y_send_sems.at[c],
                recv_sem=y_recv_sems.at[c],
                device_id=(my_x, other_y),
                device_id_type=pl.DeviceIdType.MESH,
            )
            recv.wait_recv()

        for s in x_sends:
            s.wait_send()
        for s in y_sends:
            s.wait_send()
        local_copy.wait()

    return pl.pallas_call(
        body,
        out_shape=jax.ShapeDtypeStruct((2 * m, n), jnp.bfloat16),
        in_specs=[pl.BlockSpec(memory_space=pl.ANY)],
        out_specs=pl.BlockSpec(memory_space=pl.ANY),
        scratch_shapes=[
            pltpu.SemaphoreType.DMA,
            pltpu.SemaphoreType.DMA((K,)),
            pltpu.SemaphoreType.DMA((K,)),
            pltpu.SemaphoreType.DMA((K,)),
            pltpu.SemaphoreType.DMA((K,)),
        ],
        compiler_params=pltpu.CompilerParams(collective_id=0),
    )(x)


# device time: 539400 ns/iter; 4.0615x vs baseline; 4.0615x over previous
import jax
import jax.numpy as jnp
from jax import lax
from jax.experimental import pallas as pl
from jax.experimental.pallas import tpu as pltpu

K = 16
B = 4


def kernel(x):
    x = x.astype(jnp.bfloat16)
    m, n = x.shape
    h = m // 2
    r = h // K
    rl = m // K

    def body(x_ref, out_ref, vmem, in_sems, out_sems,
             x_send_sems, x_recv_sems, y_send_sems, y_recv_sems):
        my_x = lax.axis_index("x")
        my_y = lax.axis_index("y")
        other_x = 1 - my_x
        other_y = 1 - my_y
        base_mine = my_x * m
        base_rem = other_x * m

        barrier_sem = pltpu.get_barrier_semaphore()
        for nbr in [(other_x, my_y), (my_x, other_y)]:
            pl.semaphore_signal(
                barrier_sem, inc=1,
                device_id=nbr, device_id_type=pl.DeviceIdType.MESH,
            )
        pl.semaphore_wait(barrier_sem, 2)

        ins = [None] * K
        outs = [None] * K
        for c in range(B):
            ins[c] = pltpu.make_async_copy(
                x_ref.at[pl.ds(c * rl, rl)], vmem.at[c % B], in_sems.at[c % B]
            )
            ins[c].start()

        x_sends = []
        for c in range(K):
            off = my_y * h + c * r
            s = pltpu.make_async_remote_copy(
                src_ref=x_ref.at[pl.ds(off, r)],
                dst_ref=out_ref.at[pl.ds(base_mine + off, r)],
                send_sem=x_send_sems.at[c],
                recv_sem=x_recv_sems.at[c],
                device_id=(other_x, my_y),
                device_id_type=pl.DeviceIdType.MESH,
            )
            s.start()
            x_sends.append(s)

        y_sends = []
        for c in range(K):
            rows = pl.ds(base_rem + my_y * h + c * r, r)
            recv = pltpu.make_async_remote_copy(
                src_ref=x_ref.at[pl.ds(c * r, r)],
                dst_ref=out_ref.at[rows],
                send_sem=x_send_sems.at[c],
                recv_sem=x_recv_sems.at[c],
                device_id=(other_x, my_y),
                device_id_type=pl.DeviceIdType.MESH,
            )
            recv.wait_recv()
            f = pltpu.make_async_remote_copy(
                src_ref=out_ref.at[rows],
                dst_ref=out_ref.at[rows],
                send_sem=y_send_sems.at[c],
                recv_sem=y_recv_sems.at[c],
                device_id=(my_x, other_y),
                device_id_type=pl.DeviceIdType.MESH,
            )
            f.start()
            y_sends.append(f)

            ins[c].wait()
            outs[c] = pltpu.make_async_copy(
                vmem.at[c % B],
                out_ref.at[pl.ds(base_mine + c * rl, rl)],
                out_sems.at[c % B],
            )
            outs[c].start()
            nxt = c + B
            if nxt < K:
                outs[c].wait()
                ins[nxt] = pltpu.make_async_copy(
                    x_ref.at[pl.ds(nxt * rl, rl)],
                    vmem.at[nxt % B],
                    in_sems.at[nxt % B],
                )
                ins[nxt].start()

        for c in range(K):
            recv = pltpu.make_async_remote_copy(
                src_ref=x_ref.at[pl.ds(c * r, r)],
                dst_ref=out_ref.at[pl.ds(base_rem + other_y * h + c * r, r)],
                send_sem=y_send_sems.at[c],
                recv_sem=y_recv_sems.at[c],
                device_id=(my_x, other_y),
                device_id_type=pl.DeviceIdType.MESH,
            )
            recv.wait_recv()

        for c in range(max(0, K - B), K):
            outs[c].wait()
        for s in x_sends:
            s.wait_send()
        for s in y_sends:
            s.wait_send()

    return pl.pallas_call(
        body,
        out_shape=jax.ShapeDtypeStruct((2 * m, n), jnp.bfloat16),
        in_specs=[pl.BlockSpec(memory_space=pl.ANY)],
        out_specs=pl.BlockSpec(memory_space=pl.ANY),
        scratch_shapes=[
            pltpu.VMEM((B, m // K, n), jnp.bfloat16),
            pltpu.SemaphoreType.DMA((B,)),
            pltpu.SemaphoreType.DMA((B,)),
            pltpu.SemaphoreType.DMA((K,)),
            pltpu.SemaphoreType.DMA((K,)),
            pltpu.SemaphoreType.DMA((K,)),
            pltpu.SemaphoreType.DMA((K,)),
        ],
        compiler_params=pltpu.CompilerParams(collective_id=0),
    )(x)


# device time: 476798 ns/iter; 4.5947x vs baseline; 1.1313x over previous
import jax
import jax.numpy as jnp
from jax import lax
from jax.experimental import pallas as pl
from jax.experimental.pallas import tpu as pltpu

HK = 16
LK = 2 * HK
S = 4
S2 = 4
LEAD = 3


def kernel(x):
    m, n = x.shape
    h = m // 2
    r = h // HK

    def body(x_ref, out_ref, stage, arena, rot, in_sems, out_sems,
             x_send_sems, x_recv_sems, y_send_sems, y_recv_sems):
        my_x = lax.axis_index("x")
        my_y = lax.axis_index("y")
        other_x = 1 - my_x
        other_y = 1 - my_y
        base_mine = my_x * m
        base_rem = other_x * m

        def chunk_rows(c):
            half = my_y if c < HK else other_y
            return half * h + (c % HK) * r

        barrier_sem = pltpu.get_barrier_semaphore()
        for nbr in [(other_x, my_y), (my_x, other_y)]:
            pl.semaphore_signal(
                barrier_sem, inc=1,
                device_id=nbr, device_id_type=pl.DeviceIdType.MESH,
            )
        pl.semaphore_wait(barrier_sem, 2)

        def stage_in(c):
            cp = pltpu.make_async_copy(
                x_ref.at[pl.ds(chunk_rows(c), r)],
                stage.at[c % S],
                in_sems.at[c % S],
            )
            cp.start()
            return cp

        def recv_step(c):
            rows = pl.ds(base_rem + my_y * h + c * r, r)
            recv = pltpu.make_async_remote_copy(
                src_ref=arena.at[c],
                dst_ref=out_ref.at[rows],
                send_sem=x_send_sems.at[c],
                recv_sem=x_recv_sems.at[c],
                device_id=(other_x, my_y),
                device_id_type=pl.DeviceIdType.MESH,
            )
            recv.wait_recv()
            f = pltpu.make_async_remote_copy(
                src_ref=out_ref.at[rows],
                dst_ref=out_ref.at[rows],
                send_sem=y_send_sems.at[c],
                recv_sem=y_recv_sems.at[c],
                device_id=(my_x, other_y),
                device_id_type=pl.DeviceIdType.MESH,
            )
            f.start()
            return f

        ins = [None] * LK
        outs = [None] * LK
        x_sends = []
        y_sends = []
        for c in range(S):
            ins[c] = stage_in(c)

        for c in range(LK):
            ins[c].wait()
            if c < HK:
                arena[c] = stage[c % S].astype(jnp.bfloat16)
                src = arena.at[c]
            else:
                j = c - HK
                if j >= S2:
                    outs[HK + j - S2].wait()
                rot[j % S2] = stage[c % S].astype(jnp.bfloat16)
                src = rot.at[j % S2]
            nxt = c + S
            if nxt < LK:
                ins[nxt] = stage_in(nxt)
            outs[c] = pltpu.make_async_copy(
                src, out_ref.at[pl.ds(base_mine + chunk_rows(c), r)],
                out_sems.at[c],
            )
            outs[c].start()
            if c < HK:
                s = pltpu.make_async_remote_copy(
                    src_ref=arena.at[c],
                    dst_ref=out_ref.at[pl.ds(base_mine + my_y * h + c * r, r)],
                    send_sem=x_send_sems.at[c],
                    recv_sem=x_recv_sems.at[c],
                    device_id=(other_x, my_y),
                    device_id_type=pl.DeviceIdType.MESH,
                )
                s.start()
                x_sends.append(s)
            if LEAD <= c < HK + LEAD:
                y_sends.append(recv_step(c - LEAD))

        for c in range(HK):
            recv = pltpu.make_async_remote_copy(
                src_ref=arena.at[c],
                dst_ref=out_ref.at[pl.ds(base_rem + other_y * h + c * r, r)],
                send_sem=y_send_sems.at[c],
                recv_sem=y_recv_sems.at[c],
                device_id=(my_x, other_y),
                device_id_type=pl.DeviceIdType.MESH,
            )
            recv.wait_recv()

        for c in range(LK - S2, LK):
            outs[c].wait()
        for c in range(HK):
            outs[c].wait()
        for s in x_sends:
            s.wait_send()
        for s in y_sends:
            s.wait_send()

    return pl.pallas_call(
        body,
        out_shape=jax.ShapeDtypeStruct((2 * m, n), jnp.bfloat16),
        in_specs=[pl.BlockSpec(memory_space=pl.ANY)],
        out_specs=pl.BlockSpec(memory_space=pl.ANY),
        scratch_shapes=[
            pltpu.VMEM((S, h // HK, n), jnp.float32),
            pltpu.VMEM((HK, h // HK, n), jnp.bfloat16),
            pltpu.VMEM((S2, h // HK, n), jnp.bfloat16),
            pltpu.SemaphoreType.DMA((S,)),
            pltpu.SemaphoreType.DMA((LK,)),
            pltpu.SemaphoreType.DMA((HK,)),
            pltpu.SemaphoreType.DMA((HK,)),
            pltpu.SemaphoreType.DMA((HK,)),
            pltpu.SemaphoreType.DMA((HK,)),
        ],
        compiler_params=pltpu.CompilerParams(
            collective_id=0, vmem_limit_bytes=100 * 1024 * 1024
        ),
    )(x)


# device time: 476153 ns/iter; 4.6009x vs baseline; 1.0014x over previous
import jax
import jax.numpy as jnp
from jax import lax
from jax.experimental import pallas as pl
from jax.experimental.pallas import tpu as pltpu

HK = 16
LK = 2 * HK
S = 3
S2 = 2
SB = 6
LEAD = 3


def kernel(x):
    m, n = x.shape
    h = m // 2
    r = h // HK

    def body(x_ref, out_ref, stage, arena, rot, xarena, in_sems, out_sems,
             out2_sems, x_send_sems, x_recv_sems, y_send_sems, y_recv_sems):
        my_x = lax.axis_index("x")
        my_y = lax.axis_index("y")
        other_x = 1 - my_x
        other_y = 1 - my_y
        base_mine = my_x * m
        base_rem = other_x * m

        def chunk_rows(c):
            half = my_y if c < HK else other_y
            return half * h + (c % HK) * r

        barrier_sem = pltpu.get_barrier_semaphore()
        for nbr in [(other_x, my_y), (my_x, other_y)]:
            pl.semaphore_signal(
                barrier_sem, inc=1,
                device_id=nbr, device_id_type=pl.DeviceIdType.MESH,
            )
        pl.semaphore_wait(barrier_sem, 2)

        def stage_in(c):
            cp = pltpu.make_async_copy(
                x_ref.at[pl.ds(chunk_rows(c), r)],
                stage.at[c % S],
                in_sems.at[c % S],
            )
            cp.start()
            return cp

        def recv_step(c):
            rows = pl.ds(base_rem + my_y * h + c * r, r)
            recv = pltpu.make_async_remote_copy(
                src_ref=xarena.at[c],
                dst_ref=xarena.at[c],
                send_sem=x_send_sems.at[c],
                recv_sem=x_recv_sems.at[c],
                device_id=(other_x, my_y),
                device_id_type=pl.DeviceIdType.MESH,
            )
            recv.wait_recv()
            f = pltpu.make_async_remote_copy(
                src_ref=xarena.at[c],
                dst_ref=out_ref.at[rows],
                send_sem=y_send_sems.at[c],
                recv_sem=y_recv_sems.at[c],
                device_id=(my_x, other_y),
                device_id_type=pl.DeviceIdType.MESH,
            )
            f.start()
            lc = pltpu.make_async_copy(
                xarena.at[c], out_ref.at[rows], out2_sems.at[c]
            )
            lc.start()
            return f, lc

        ins = [None] * LK
        outs = [None] * LK
        x_sends = []
        y_sends = []
        x_places = []
        for c in range(S):
            ins[c] = stage_in(c)

        for c in range(LK):
            ins[c].wait()
            if c < HK:
                if c >= SB:
                    x_sends[c - SB].wait_send()
                    outs[c - SB].wait()
                arena[c % SB] = stage[c % S].astype(jnp.bfloat16)
                src = arena.at[c % SB]
            else:
                j = c - HK
                if j >= S2:
                    outs[HK + j - S2].wait()
                rot[j % S2] = stage[c % S].astype(jnp.bfloat16)
                src = rot.at[j % S2]
            nxt = c + S
            if nxt < LK:
                ins[nxt] = stage_in(nxt)
            outs[c] = pltpu.make_async_copy(
                src, out_ref.at[pl.ds(base_mine + chunk_rows(c), r)],
                out_sems.at[c],
            )
            outs[c].start()
            if c < HK:
                s = pltpu.make_async_remote_copy(
                    src_ref=arena.at[c % SB],
                    dst_ref=xarena.at[c],
                    send_sem=x_send_sems.at[c],
                    recv_sem=x_recv_sems.at[c],
                    device_id=(other_x, my_y),
                    device_id_type=pl.DeviceIdType.MESH,
                )
                s.start()
                x_sends.append(s)
            if LEAD <= c < HK + LEAD:
                f, lc = recv_step(c - LEAD)
                y_sends.append(f)
                x_places.append(lc)

        for c in range(HK):
            recv = pltpu.make_async_remote_copy(
                src_ref=xarena.at[c],
                dst_ref=out_ref.at[pl.ds(base_rem + other_y * h + c * r, r)],
                send_sem=y_send_sems.at[c],
                recv_sem=y_recv_sems.at[c],
                device_id=(my_x, other_y),
                device_id_type=pl.DeviceIdType.MESH,
            )
            recv.wait_recv()

        for c in range(LK - S2, LK):
            outs[c].wait()
        for c in range(HK - SB, HK):
            outs[c].wait()
        for lc in x_places:
            lc.wait()
        for s in x_sends[HK - SB:]:
            s.wait_send()
        for s in y_sends:
            s.wait_send()

    return pl.pallas_call(
        body,
        out_shape=jax.ShapeDtypeStruct((2 * m, n), jnp.bfloat16),
        in_specs=[pl.BlockSpec(memory_space=pl.ANY)],
        out_specs=pl.BlockSpec(memory_space=pl.ANY),
        scratch_shapes=[
            pltpu.VMEM((S, h // HK, n), jnp.float32),
            pltpu.VMEM((SB, h // HK, n), jnp.bfloat16),
            pltpu.VMEM((S2, h // HK, n), jnp.bfloat16),
            pltpu.VMEM((HK, h // HK, n), jnp.bfloat16),
            pltpu.SemaphoreType.DMA((S,)),
            pltpu.SemaphoreType.DMA((LK,)),
            pltpu.SemaphoreType.DMA((HK,)),
            pltpu.SemaphoreType.DMA((HK,)),
            pltpu.SemaphoreType.DMA((HK,)),
            pltpu.SemaphoreType.DMA((HK,)),
            pltpu.SemaphoreType.DMA((HK,)),
        ],
        compiler_params=pltpu.CompilerParams(
            collective_id=0, vmem_limit_bytes=100 * 1024 * 1024
        ),
    )(x)
